# baseline (speedup 1.0000x reference)
import jax
import jax.numpy as jnp
from jax.experimental import pallas as pl
from jax.experimental.pallas import tpu as pltpu

B, W, R, M = 256, 4, 8, 8192
LN = 128
SL = M // LN
BB = 16


def _freeness_kernel(fg_ref, ww_ref, rw_ref, prev_ref, out_ref):
    for b in range(BB):
        prev = prev_ref[b]
        keep = 1.0 - ww_ref[b, 0]
        for w in range(1, W):
            keep = keep * (1.0 - ww_ref[b, w])
        usage = prev + (1.0 - prev) * (1.0 - keep)
        phi = 1.0 - fg_ref[b, 0] * rw_ref[b, 0]
        for r in range(1, R):
            phi = phi * (1.0 - fg_ref[b, r] * rw_ref[b, r])
        out_ref[b] = usage * phi


def kernel(write_weights, free_gate, read_weights, prev_usage):
    wwr = write_weights.reshape(B, W, SL, LN)
    rwr = read_weights.reshape(B, R, SL, LN)
    prevr = prev_usage.reshape(B, SL, LN)
    grid = (B // BB,)
    out = pl.pallas_call(
        _freeness_kernel,
        grid=grid,
        in_specs=[
            pl.BlockSpec((BB, R), lambda i: (i, 0), memory_space=pltpu.SMEM),
            pl.BlockSpec((BB, W, SL, LN), lambda i: (i, 0, 0, 0)),
            pl.BlockSpec((BB, R, SL, LN), lambda i: (i, 0, 0, 0)),
            pl.BlockSpec((BB, SL, LN), lambda i: (i, 0, 0)),
        ],
        out_specs=pl.BlockSpec((BB, SL, LN), lambda i: (i, 0, 0)),
        out_shape=jax.ShapeDtypeStruct((B, SL, LN), jnp.float32),
    )(free_gate, wwr, rwr, prevr)
    return out.reshape(B, M)


# speedup vs baseline: 3.5766x; 3.5766x over previous
import jax
import jax.numpy as jnp
from jax.experimental import pallas as pl
from jax.experimental.pallas import tpu as pltpu

B, W, R, M = 256, 4, 8, 8192
BB = 16


def _freeness_kernel(fg_ref, ww_ref, rw_ref, prev_ref, out_ref):
    for b in range(BB):
        prev = prev_ref[b]
        keep = 1.0 - ww_ref[b, 0]
        for w in range(1, W):
            keep = keep * (1.0 - ww_ref[b, w])
        usage = prev + (1.0 - prev) * (1.0 - keep)
        phi = 1.0 - fg_ref[b, 0] * rw_ref[b, 0]
        for r in range(1, R):
            phi = phi * (1.0 - fg_ref[b, r] * rw_ref[b, r])
        out_ref[b] = usage * phi


def kernel(write_weights, free_gate, read_weights, prev_usage):
    grid = (B // BB,)
    return pl.pallas_call(
        _freeness_kernel,
        grid=grid,
        in_specs=[
            pl.BlockSpec((BB, R), lambda i: (i, 0), memory_space=pltpu.SMEM),
            pl.BlockSpec((BB, W, M), lambda i: (i, 0, 0)),
            pl.BlockSpec((BB, R, M), lambda i: (i, 0, 0)),
            pl.BlockSpec((BB, M), lambda i: (i, 0)),
        ],
        out_specs=pl.BlockSpec((BB, M), lambda i: (i, 0)),
        out_shape=jax.ShapeDtypeStruct((B, M), jnp.float32),
    )(free_gate, write_weights, read_weights, prev_usage)
